# baseline (speedup 1.0000x reference)
import functools

import jax
import jax.numpy as jnp
from jax import lax
from jax.experimental import pallas as pl
from jax.experimental.pallas import tpu as pltpu
from jax.experimental.pallas import tpu_sc as plsc

EMBEDDING_DIM = 50
CONTEXT_SIZE = 5
NUM_CORES = 2
NUM_SUBCORES = 16
NUM_WORKERS = NUM_CORES * NUM_SUBCORES
CHUNK = 128


@functools.partial(jax.jit, static_argnames=("n_chunks",))
def _sc_gather3(emb, p_emb, s_emb, gi, pi, si, n_chunks):
    n_rows = NUM_WORKERS * n_chunks * CHUNK
    out_t = [jax.ShapeDtypeStruct((n_rows, EMBEDDING_DIM), jnp.float32)] * 3
    mesh = plsc.VectorSubcoreMesh(core_axis_name="c", subcore_axis_name="s")

    @functools.partial(
        pl.kernel,
        mesh=mesh,
        out_type=out_t,
        compiler_params=pltpu.CompilerParams(use_tc_tiling_on_sc=False),
        scratch_types=[
            pltpu.VMEM((n_chunks, CHUNK), jnp.int32),
            pltpu.VMEM((n_chunks, CHUNK), jnp.int32),
            pltpu.VMEM((n_chunks, CHUNK), jnp.int32),
            pltpu.VMEM((CHUNK, EMBEDDING_DIM), jnp.float32),
            pltpu.VMEM((CHUNK, EMBEDDING_DIM), jnp.float32),
            pltpu.VMEM((CHUNK, EMBEDDING_DIM), jnp.float32),
            pltpu.SemaphoreType.DMA,
            pltpu.SemaphoreType.DMA,
            pltpu.SemaphoreType.DMA,
        ],
    )
    def k(emb_h, p_h, s_h, gi_h, pi_h, si_h, go_h, po_h, so_h,
          gi_v, pi_v, si_v, gbuf, pbuf, sbuf, sem0, sem1, sem2):
        wid = lax.axis_index("s") * NUM_CORES + lax.axis_index("c")
        pltpu.sync_copy(gi_h.at[wid], gi_v)
        pltpu.sync_copy(pi_h.at[wid], pi_v)
        pltpu.sync_copy(si_h.at[wid], si_v)
        row0 = wid * (n_chunks * CHUNK)

        def body(j, carry):
            cp0 = pltpu.async_copy(emb_h.at[gi_v.at[j]], gbuf, sem0)
            cp1 = pltpu.async_copy(p_h.at[pi_v.at[j]], pbuf, sem1)
            cp2 = pltpu.async_copy(s_h.at[si_v.at[j]], sbuf, sem2)
            cp0.wait()
            cp1.wait()
            cp2.wait()
            base = row0 + j * CHUNK
            pltpu.sync_copy(gbuf, go_h.at[pl.ds(base, CHUNK)])
            pltpu.sync_copy(pbuf, po_h.at[pl.ds(base, CHUNK)])
            pltpu.sync_copy(sbuf, so_h.at[pl.ds(base, CHUNK)])
            return carry

        lax.fori_loop(0, n_chunks, body, 0)

    return k(emb, p_emb, s_emb, gi, pi, si)


def _mlp_body(g_ref, p_ref, s_ref, w1_ref, b1_ref, w2_ref, b2_ref, o_ref):
    x = g_ref[...] + p_ref[...] + s_ref[...]
    h = jnp.dot(x, w1_ref[...], preferred_element_type=jnp.float32)
    h = jnp.maximum(h + b1_ref[...], 0.0)
    o = jnp.dot(h, w2_ref[...], preferred_element_type=jnp.float32)
    o = o + b2_ref[...]
    m = jnp.max(o, axis=1, keepdims=True)
    e = jnp.exp(o - m)
    lse = jnp.log(jnp.sum(e, axis=1, keepdims=True))
    o_ref[...] = (o - m) - lse


@jax.jit
def _tc_mlp(G, P, S, W1, b1, W2, b2):
    B, F = G.shape
    blk = 1024
    n_tags = W2.shape[1]
    hidden = W1.shape[1]
    grid = (B // blk,)
    return pl.pallas_call(
        _mlp_body,
        grid=grid,
        in_specs=[
            pl.BlockSpec((blk, F), lambda i: (i, 0)),
            pl.BlockSpec((blk, F), lambda i: (i, 0)),
            pl.BlockSpec((blk, F), lambda i: (i, 0)),
            pl.BlockSpec((F, hidden), lambda i: (0, 0)),
            pl.BlockSpec((1, hidden), lambda i: (0, 0)),
            pl.BlockSpec((hidden, n_tags), lambda i: (0, 0)),
            pl.BlockSpec((1, n_tags), lambda i: (0, 0)),
        ],
        out_specs=pl.BlockSpec((blk, n_tags), lambda i: (i, 0)),
        out_shape=jax.ShapeDtypeStruct((B, n_tags), jnp.float32),
    )(G, P, S, W1, b1, W2, b2)


def kernel(inputs, p_inputs, s_inputs, emb, p_emb, s_emb, W1, b1, W2, b2):
    ctx, batch = inputs.shape
    n_rows = batch * ctx
    n_chunks = n_rows // (NUM_WORKERS * CHUNK)

    def prep(ix):
        return ix.T.reshape(NUM_WORKERS, n_chunks, CHUNK).astype(jnp.int32)

    G, P, S = _sc_gather3(emb, p_emb, s_emb,
                          prep(inputs), prep(p_inputs), prep(s_inputs),
                          n_chunks)
    feat = ctx * emb.shape[1]
    out = _tc_mlp(G.reshape(batch, feat), P.reshape(batch, feat),
                  S.reshape(batch, feat),
                  W1, b1.reshape(1, -1), W2, b2.reshape(1, -1))
    return out
